# Initial kernel scaffold; baseline (speedup 1.0000x reference)
import jax
import jax.numpy as jnp
from jax.experimental import pallas as pl


def kernel(x, edge_index, graph_ids, eps, W1, b1, g1, be1, W2, b2, bn_g, bn_b, Wp, bp):
    raise NotImplementedError("write your pallas kernel here")


# speedup vs baseline: 1.7536x; 1.7536x over previous
import functools

import jax
import jax.numpy as jnp
from jax import lax
from jax.experimental import pallas as pl
from jax.experimental.pallas import tpu as pltpu
from jax.experimental.pallas import tpu_sc as plsc

N = 10000
E = 320000
D = 128
H = 128
OUT = 16
B = 128
L = 5

NC = 2
NS = 16
NW = NC * NS

K = 128
CH = 80
EPT = CH * K
E_PAD = EPT * NW
HALF = 5120
NPH = 5248
ZPT = NPH // NS

_sc_mesh = plsc.VectorSubcoreMesh(
    core_axis_name="c", subcore_axis_name="s", num_cores=NC, num_subcores=NS)


@functools.partial(
    pl.kernel,
    out_type=jax.ShapeDtypeStruct((NC, N, D), jnp.float32),
    mesh=_sc_mesh,
    scratch_types=[
        pltpu.VMEM((CH, K), jnp.int32),
        pltpu.VMEM((CH, K), jnp.int32),
        pltpu.VMEM((CH, K), jnp.int32),
        pltpu.VMEM((K, D), jnp.float32),
        pltpu.VMEM((K, D), jnp.float32),
        pltpu.VMEM_SHARED((NPH, D), jnp.float32),
        pltpu.SemaphoreType.DMA,
        pltpu.SemaphoreType.DMA,
    ],
)
def _sc_spmm(h_hbm, src_hbm, dst0_hbm, dst1_hbm, out_hbm,
             src_v, dst0_v, dst1_v, rows_a, rows_b, pooled, sem_a, sem_b):
    c = lax.axis_index("c")
    s = lax.axis_index("s")
    wid = s * NC + c

    pltpu.sync_copy(src_hbm.at[wid], src_v)
    pltpu.sync_copy(dst0_hbm.at[wid], dst0_v)
    pltpu.sync_copy(dst1_hbm.at[wid], dst1_v)

    zvec = jnp.zeros((16,), jnp.float32)

    def zfill(i, carry):
        for k in range(D // 16):
            rows_a[i, pl.ds(k * 16, 16)] = zvec
        return carry

    def run_pass(p, dst_v):
        lax.fori_loop(0, K, zfill, 0)
        pltpu.sync_copy(rows_a, pooled.at[pl.ds(s * ZPT, K)])
        pltpu.sync_copy(rows_a, pooled.at[pl.ds(s * ZPT + K, K)])
        pltpu.sync_copy(rows_a.at[pl.ds(0, ZPT - 2 * K)],
                        pooled.at[pl.ds(s * ZPT + 2 * K, ZPT - 2 * K)])
        plsc.subcore_barrier()

        pltpu.async_copy(h_hbm.at[src_v.at[0]], rows_a, sem_a)

        def body(t, carry):
            a_chunk = 2 * t
            b_chunk = 2 * t + 1
            next_a = (2 * t + 2) % CH
            pltpu.make_async_copy(h_hbm.at[src_v.at[a_chunk]], rows_a,
                                  sem_a).wait()
            pltpu.async_copy(h_hbm.at[src_v.at[b_chunk]], rows_b, sem_b)
            pltpu.sync_copy(rows_a, pooled.at[dst_v.at[a_chunk]], add=True)
            pltpu.async_copy(h_hbm.at[src_v.at[next_a]], rows_a, sem_a)
            pltpu.make_async_copy(h_hbm.at[src_v.at[b_chunk]], rows_b,
                                  sem_b).wait()
            pltpu.sync_copy(rows_b, pooled.at[dst_v.at[b_chunk]], add=True)
            return carry

        lax.fori_loop(0, CH // 2, body, 0)
        pltpu.make_async_copy(h_hbm.at[src_v.at[0]], rows_a, sem_a).wait()

        plsc.subcore_barrier()

        if p == 0:
            rpt = HALF // NS
            pltpu.sync_copy(pooled.at[pl.ds(s * rpt, rpt)],
                            out_hbm.at[c, pl.ds(s * rpt, rpt)])
        else:
            rem = N - HALF
            rpt = 312

            @pl.when(s < NS - 1)
            def _():
                pltpu.sync_copy(pooled.at[pl.ds(s * rpt, rpt)],
                                out_hbm.at[c, pl.ds(HALF + s * rpt, rpt)])

            @pl.when(s == NS - 1)
            def _():
                pltpu.sync_copy(
                    pooled.at[pl.ds((NS - 1) * rpt, rem - (NS - 1) * rpt)],
                    out_hbm.at[c, pl.ds(HALF + (NS - 1) * rpt,
                                        rem - (NS - 1) * rpt)])

        plsc.subcore_barrier()

    run_pass(0, dst0_v)
    run_pass(1, dst1_v)


def _bn_relu(z, gamma, beta):
    mean = jnp.mean(z, axis=0, keepdims=True)
    var = jnp.mean((z - mean) ** 2, axis=0, keepdims=True)
    return jnp.maximum((z - mean) * lax.rsqrt(var + 1e-5) * gamma + beta, 0.0)


def _mlp_core(part_ref, h_ref, eps_ref, w1_ref, b1_ref, g1_ref, be1_ref,
              w2_ref, b2_ref, bng_ref, bnb_ref):
    h = h_ref[...]
    p = part_ref[0] + part_ref[1] + (1.0 + eps_ref[0, 0]) * h
    z = jnp.dot(p, w1_ref[...], preferred_element_type=jnp.float32) + b1_ref[...]
    z = _bn_relu(z, g1_ref[...], be1_ref[...])
    z = jnp.dot(z, w2_ref[...], preferred_element_type=jnp.float32) + b2_ref[...]
    return _bn_relu(z, bng_ref[...], bnb_ref[...])


def _layer0_body(part_ref, h_ref, eps_ref, w1_ref, b1_ref, g1_ref, be1_ref,
                 w2_ref, b2_ref, bng_ref, bnb_ref, wp0_ref, wp1_ref,
                 h_out_ref, acc_out_ref):
    hn = _mlp_core(part_ref, h_ref, eps_ref, w1_ref, b1_ref, g1_ref, be1_ref,
                   w2_ref, b2_ref, bng_ref, bnb_ref)
    h_out_ref[...] = hn
    acc_out_ref[...] = (
        jnp.dot(h_ref[...], wp0_ref[...], preferred_element_type=jnp.float32)
        + jnp.dot(hn, wp1_ref[...], preferred_element_type=jnp.float32))


def _layer_mid_body(part_ref, h_ref, eps_ref, w1_ref, b1_ref, g1_ref, be1_ref,
                    w2_ref, b2_ref, bng_ref, bnb_ref, wp_ref, acc_ref,
                    h_out_ref, acc_out_ref):
    hn = _mlp_core(part_ref, h_ref, eps_ref, w1_ref, b1_ref, g1_ref, be1_ref,
                   w2_ref, b2_ref, bng_ref, bnb_ref)
    h_out_ref[...] = hn
    acc_out_ref[...] = acc_ref[...] + jnp.dot(
        hn, wp_ref[...], preferred_element_type=jnp.float32)


def _layer_last_body(part_ref, h_ref, eps_ref, w1_ref, b1_ref, g1_ref, be1_ref,
                     w2_ref, b2_ref, bng_ref, bnb_ref, wp_ref, acc_ref,
                     gid_ref, bp_ref, out_ref):
    hn = _mlp_core(part_ref, h_ref, eps_ref, w1_ref, b1_ref, g1_ref, be1_ref,
                   w2_ref, b2_ref, bng_ref, bnb_ref)
    acc = acc_ref[...] + jnp.dot(hn, wp_ref[...],
                                 preferred_element_type=jnp.float32)
    ids = gid_ref[...]
    iota = lax.broadcasted_iota(jnp.int32, (N, B), 1)
    onehot = jnp.where(ids == iota, 1.0, 0.0)
    out = lax.dot_general(onehot, acc, (((0,), (0,)), ((), ())),
                          preferred_element_type=jnp.float32)
    out_ref[...] = out + jnp.sum(bp_ref[...], axis=0, keepdims=True)


_layer0_call = pl.pallas_call(
    _layer0_body,
    out_shape=[jax.ShapeDtypeStruct((N, H), jnp.float32),
               jax.ShapeDtypeStruct((N, OUT), jnp.float32)],
)

_layer_mid_call = pl.pallas_call(
    _layer_mid_body,
    out_shape=[jax.ShapeDtypeStruct((N, H), jnp.float32),
               jax.ShapeDtypeStruct((N, OUT), jnp.float32)],
)

_layer_last_call = pl.pallas_call(
    _layer_last_body,
    out_shape=jax.ShapeDtypeStruct((B, OUT), jnp.float32),
)


def kernel(x, edge_index, graph_ids, eps, W1, b1, g1, be1, W2, b2,
           bn_g, bn_b, Wp, bp):
    src = edge_index[1].astype(jnp.int32)
    dst = edge_index[0].astype(jnp.int32)
    pad = E_PAD - E
    src_p = jnp.concatenate([src, jnp.zeros((pad,), jnp.int32)]).reshape(NW, CH, K)
    dstg = jnp.concatenate([dst, jnp.full((pad,), 2 * HALF, jnp.int32)])
    dst0 = jnp.where(dstg < HALF, dstg, HALF).reshape(NW, CH, K)
    dst1 = jnp.where((dstg >= HALF) & (dstg < 2 * HALF), dstg - HALF,
                     HALF).reshape(NW, CH, K)
    gid = graph_ids.astype(jnp.int32).reshape(N, 1)

    h = x
    acc = None
    for l in range(L - 1):
        part = _sc_spmm(h, src_p, dst0, dst1)
        eps_l = eps[l].reshape(1, 1)
        params = (eps_l, W1[l], b1[l].reshape(1, H), g1[l].reshape(1, H),
                  be1[l].reshape(1, H), W2[l], b2[l].reshape(1, H),
                  bn_g[l].reshape(1, H), bn_b[l].reshape(1, H))
        if l == 0:
            h, acc = _layer0_call(part, h, *params, Wp[0], Wp[1])
        elif l < L - 2:
            h, acc = _layer_mid_call(part, h, *params, Wp[l + 1], acc)
        else:
            return _layer_last_call(part, h, *params, Wp[l + 1], acc, gid, bp)
